# baseline (speedup 1.0000x reference)
import jax
import jax.numpy as jnp
from jax.experimental import pallas as pl
from jax.experimental.pallas import tpu as pltpu

B, L, D, DO = 16, 2048, 1024, 128
TL = 256
NB = L // TL
GMAX = B * NB


def _body(bag_ref, jj_ref, cnt_ref, n_ref,
          x_ref, W1_ref, b1_ref, W2_ref, b2_ref,
          out_ref, acc_ref):
    g = pl.program_id(0)

    @pl.when(g < cnt_ref[0])
    def _():
        bg = bag_ref[g]
        jg = jj_ref[g]
        nb = n_ref[bg]

        xb = x_ref[0]
        y = jnp.dot(xb, W1_ref[...], preferred_element_type=jnp.float32)
        y = jnp.maximum(y + b1_ref[...], 0.0)
        rows = jg * TL + jax.lax.broadcasted_iota(jnp.int32, (TL, 1), 0)
        y = jnp.where(rows < nb, y, 0.0)
        s = jnp.sum(y, axis=0, keepdims=True)
        prev = jnp.where(jg == 0, jnp.zeros_like(s), acc_ref[...])
        tot = prev + s
        acc_ref[...] = tot

        @pl.when((jg + 1) * TL >= nb)
        def _():
            pooled = tot / nb.astype(jnp.float32)
            out_ref[pl.ds(bg, 1), :] = (
                jnp.dot(pooled, W2_ref[...], preferred_element_type=jnp.float32)
                + b2_ref[...]
            )


def kernel(x, n_instances, W1, b1, W2, b2):
    n = n_instances.astype(jnp.int32)
    b1r = b1.reshape(1, D)
    b2r = b2.reshape(1, DO)

    nb = (n + TL - 1) // TL
    ends = jnp.cumsum(nb)
    total = ends[B - 1].astype(jnp.int32)
    g = jnp.minimum(jnp.arange(GMAX, dtype=jnp.int32), total - 1)
    bag_of_g = jnp.minimum(
        jnp.searchsorted(ends, g, side="right").astype(jnp.int32), B - 1)
    j_of_g = g - (ends - nb)[bag_of_g]

    grid_spec = pltpu.PrefetchScalarGridSpec(
        num_scalar_prefetch=4,
        grid=(GMAX,),
        in_specs=[
            pl.BlockSpec(
                (1, TL, D),
                lambda g, bag, jj, cnt, nn: (bag[g], jj[g], 0),
            ),
            pl.BlockSpec((D, D), lambda *_: (0, 0)),
            pl.BlockSpec((1, D), lambda *_: (0, 0)),
            pl.BlockSpec((D, DO), lambda *_: (0, 0)),
            pl.BlockSpec((1, DO), lambda *_: (0, 0)),
        ],
        out_specs=pl.BlockSpec((B, DO), lambda *_: (0, 0)),
        scratch_shapes=[pltpu.VMEM((1, D), jnp.float32)],
    )

    return pl.pallas_call(
        _body,
        grid_spec=grid_spec,
        out_shape=jax.ShapeDtypeStruct((B, DO), jnp.float32),
        compiler_params=pltpu.CompilerParams(
            dimension_semantics=("arbitrary",),
        ),
    )(bag_of_g, j_of_g, total.reshape(1), n, x, W1, b1r, W2, b2r)


# speedup vs baseline: 1.9439x; 1.9439x over previous
import jax
import jax.numpy as jnp
from jax.experimental import pallas as pl
from jax.experimental.pallas import tpu as pltpu

B, L, D, DO = 16, 2048, 1024, 128
TLI = 256
NBI = L // TLI


def _body(n_ref, x_ref, W1_ref, b1_ref, W2_ref, b2_ref, out_ref):
    b = pl.program_id(0)
    nb = n_ref[b]
    jmax = (nb + TLI - 1) // TLI

    def inner(j, acc):
        xb = x_ref[0, pl.ds(j * TLI, TLI), :]
        y = jnp.dot(xb, W1_ref[...], preferred_element_type=jnp.float32)
        y = jnp.maximum(y + b1_ref[...], 0.0)
        rows = j * TLI + jax.lax.broadcasted_iota(jnp.int32, (TLI, 1), 0)
        y = jnp.where(rows < nb, y, 0.0)
        return acc + jnp.sum(y, axis=0, keepdims=True)

    acc = jax.lax.fori_loop(
        0, jmax, inner, jnp.zeros((1, D), jnp.float32))

    pooled = acc / nb.astype(jnp.float32)
    out_ref[pl.ds(b, 1), :] = (
        jnp.dot(pooled, W2_ref[...], preferred_element_type=jnp.float32)
        + b2_ref[...]
    )


def kernel(x, n_instances, W1, b1, W2, b2):
    n = n_instances.astype(jnp.int32)
    b1r = b1.reshape(1, D)
    b2r = b2.reshape(1, DO)

    grid_spec = pltpu.PrefetchScalarGridSpec(
        num_scalar_prefetch=1,
        grid=(B,),
        in_specs=[
            pl.BlockSpec((1, L, D), lambda b, nn: (b, 0, 0)),
            pl.BlockSpec((D, D), lambda *_: (0, 0)),
            pl.BlockSpec((1, D), lambda *_: (0, 0)),
            pl.BlockSpec((D, DO), lambda *_: (0, 0)),
            pl.BlockSpec((1, DO), lambda *_: (0, 0)),
        ],
        out_specs=pl.BlockSpec((B, DO), lambda *_: (0, 0)),
        scratch_shapes=[],
    )

    return pl.pallas_call(
        _body,
        grid_spec=grid_spec,
        out_shape=jax.ShapeDtypeStruct((B, DO), jnp.float32),
        compiler_params=pltpu.CompilerParams(
            dimension_semantics=("arbitrary",),
        ),
    )(n, x, W1, b1r, W2, b2r)


# speedup vs baseline: 2.2324x; 1.1484x over previous
import jax
import jax.numpy as jnp
from jax.experimental import pallas as pl
from jax.experimental.pallas import tpu as pltpu

B, L, D, DO = 16, 2048, 1024, 128
TLI = 256
NSLAB = 3


def _body(n_ref, x_hbm, W1_ref, b1_ref, W2_ref, b2_ref, out_ref, buf, sems):
    def dma(b):
        slot = b % NSLAB
        return pltpu.make_async_copy(
            x_hbm.at[b], buf.at[slot], sems.at[slot])

    for b in range(NSLAB - 1):
        dma(b).start()

    for b in range(B):
        if b + NSLAB - 1 < B:
            dma(b + NSLAB - 1).start()
        dma(b).wait()
        slot = b % NSLAB

        nb = n_ref[b]
        jmax = (nb + TLI - 1) // TLI

        def inner(j, acc, slot=slot, nb=nb):
            xb = buf[slot, pl.ds(j * TLI, TLI), :]
            y = jnp.dot(xb, W1_ref[...], preferred_element_type=jnp.float32)
            y = jnp.maximum(y + b1_ref[...], 0.0)
            rows = j * TLI + jax.lax.broadcasted_iota(jnp.int32, (TLI, 1), 0)
            y = jnp.where(rows < nb, y, 0.0)
            return acc + jnp.sum(y, axis=0, keepdims=True)

        acc = jax.lax.fori_loop(
            0, jmax, inner, jnp.zeros((1, D), jnp.float32))

        pooled = acc / nb.astype(jnp.float32)
        out_ref[pl.ds(b, 1), :] = (
            jnp.dot(pooled, W2_ref[...], preferred_element_type=jnp.float32)
            + b2_ref[...]
        )


def kernel(x, n_instances, W1, b1, W2, b2):
    n = n_instances.astype(jnp.int32)
    b1r = b1.reshape(1, D)
    b2r = b2.reshape(1, DO)

    grid_spec = pltpu.PrefetchScalarGridSpec(
        num_scalar_prefetch=1,
        grid=(1,),
        in_specs=[
            pl.BlockSpec(memory_space=pl.ANY),
            pl.BlockSpec((D, D), lambda *_: (0, 0)),
            pl.BlockSpec((1, D), lambda *_: (0, 0)),
            pl.BlockSpec((D, DO), lambda *_: (0, 0)),
            pl.BlockSpec((1, DO), lambda *_: (0, 0)),
        ],
        out_specs=pl.BlockSpec((B, DO), lambda *_: (0, 0)),
        scratch_shapes=[
            pltpu.VMEM((NSLAB, L, D), jnp.float32),
            pltpu.SemaphoreType.DMA((NSLAB,)),
        ],
    )

    return pl.pallas_call(
        _body,
        grid_spec=grid_spec,
        out_shape=jax.ShapeDtypeStruct((B, DO), jnp.float32),
        compiler_params=pltpu.CompilerParams(
            dimension_semantics=("arbitrary",),
        ),
    )(n, x, W1, b1r, W2, b2r)
